# Initial kernel scaffold; baseline (speedup 1.0000x reference)
import jax
import jax.numpy as jnp
from jax.experimental import pallas as pl


def kernel(x, pos_table, ln_gamma, ln_beta):
    raise NotImplementedError("write your pallas kernel here")


# speedup vs baseline: 1.7552x; 1.7552x over previous
import jax
import jax.numpy as jnp
from jax.experimental import pallas as pl

_D = 1024
_B = 4
_LN_EPS = 1e-5
_TS = 256


def _ln_kernel(x_ref, pe_ref, g_ref, b_ref, o_ref):
    pe = pe_ref[...]
    g = g_ref[...]
    b = b_ref[...]
    for i in range(_B):
        sl = slice(i * _D, (i + 1) * _D)
        h = x_ref[:, sl] + pe
        mean = jnp.mean(h, axis=1, keepdims=True)
        hc = h - mean
        var = jnp.mean(hc * hc, axis=1, keepdims=True)
        o_ref[:, sl] = hc * jax.lax.rsqrt(var + _LN_EPS) * g + b


def kernel(x, pos_table, ln_gamma, ln_beta):
    S, B, D = x.shape
    x2 = x.reshape(S, B * D)
    g2 = ln_gamma.reshape(1, D)
    b2 = ln_beta.reshape(1, D)
    out = pl.pallas_call(
        _ln_kernel,
        grid=(S // _TS,),
        in_specs=[
            pl.BlockSpec((_TS, B * D), lambda s: (s, 0)),
            pl.BlockSpec((_TS, D), lambda s: (s, 0)),
            pl.BlockSpec((1, D), lambda s: (0, 0)),
            pl.BlockSpec((1, D), lambda s: (0, 0)),
        ],
        out_specs=pl.BlockSpec((_TS, B * D), lambda s: (s, 0)),
        out_shape=jax.ShapeDtypeStruct((S, B * D), x.dtype),
    )(x2, pos_table, g2, b2)
    return out.reshape(S, B, D)


# speedup vs baseline: 5.3329x; 3.0383x over previous
import jax
import jax.numpy as jnp
from jax.experimental import pallas as pl

_D = 1024
_B = 4
_LN_EPS = 1e-5
_TS = 256


def _ln_kernel(x_ref, pe_ref, g_ref, b_ref, o_ref):
    pe = pe_ref[...]
    g = g_ref[...]
    b = b_ref[...]
    h = x_ref[...] + pe[:, None, :]
    mean = jnp.mean(h, axis=-1, keepdims=True)
    hc = h - mean
    var = jnp.mean(hc * hc, axis=-1, keepdims=True)
    o_ref[...] = hc * jax.lax.rsqrt(var + _LN_EPS) * g[None] + b[None]


def kernel(x, pos_table, ln_gamma, ln_beta):
    S, B, D = x.shape
    g2 = ln_gamma.reshape(1, D)
    b2 = ln_beta.reshape(1, D)
    out = pl.pallas_call(
        _ln_kernel,
        grid=(S // _TS,),
        in_specs=[
            pl.BlockSpec((_TS, B, D), lambda s: (s, 0, 0)),
            pl.BlockSpec((_TS, D), lambda s: (s, 0)),
            pl.BlockSpec((1, D), lambda s: (0, 0)),
            pl.BlockSpec((1, D), lambda s: (0, 0)),
        ],
        out_specs=pl.BlockSpec((_TS, B, D), lambda s: (s, 0, 0)),
        out_shape=jax.ShapeDtypeStruct((S, B, D), x.dtype),
    )(x, pos_table, g2, b2)
    return out


# speedup vs baseline: 5.6609x; 1.0615x over previous
import jax
import jax.numpy as jnp
from jax.experimental import pallas as pl

_D = 1024
_B = 4
_LN_EPS = 1e-5
_TS = 512


def _ln_kernel(x_ref, pe_ref, g_ref, b_ref, o_ref):
    pe = pe_ref[...]
    g = g_ref[...]
    b = b_ref[...]
    h = x_ref[...] + pe[:, None, :]
    mean = jnp.mean(h, axis=-1, keepdims=True)
    hc = h - mean
    var = jnp.mean(hc * hc, axis=-1, keepdims=True)
    o_ref[...] = hc * jax.lax.rsqrt(var + _LN_EPS) * g[None] + b[None]


def kernel(x, pos_table, ln_gamma, ln_beta):
    S, B, D = x.shape
    g2 = ln_gamma.reshape(1, D)
    b2 = ln_beta.reshape(1, D)
    out = pl.pallas_call(
        _ln_kernel,
        grid=(S // _TS,),
        in_specs=[
            pl.BlockSpec((_TS, B, D), lambda s: (s, 0, 0)),
            pl.BlockSpec((_TS, D), lambda s: (s, 0)),
            pl.BlockSpec((1, D), lambda s: (0, 0)),
            pl.BlockSpec((1, D), lambda s: (0, 0)),
        ],
        out_specs=pl.BlockSpec((_TS, B, D), lambda s: (s, 0, 0)),
        out_shape=jax.ShapeDtypeStruct((S, B, D), x.dtype),
    )(x, pos_table, g2, b2)
    return out
